# Initial kernel scaffold; baseline (speedup 1.0000x reference)
import functools

import jax
import jax.numpy as jnp
from jax import lax
from jax.experimental import pallas as pl
from jax.experimental.pallas import tpu as pltpu
from jax.experimental.pallas import tpu_sc as plsc

_N = 10000
_E = 320000
_F = 128
_NG = 16
_NPAD = 10240
_CHUNK = 128
_NTILES = 32
_CPT = 79
_EPAD = _NTILES * _CPT * _CHUNK
_SINK = _NPAD - _N
_RPS = _NPAD // 16


def _sc_mesh():
    return plsc.VectorSubcoreMesh(core_axis_name="c", subcore_axis_name="s")


def _deg_body(dst_hbm, ones_hbm, zeros_hbm, out_hbm, acc, dst_v, ones_v):
    cid = lax.axis_index("c")
    sid = lax.axis_index("s")
    wid = sid * 2 + cid
    pltpu.sync_copy(zeros_hbm.at[pl.ds(sid * _RPS, _RPS)],
                    acc.at[pl.ds(sid * _RPS, _RPS)])
    pltpu.sync_copy(ones_hbm, ones_v)
    pltpu.sync_copy(dst_hbm.at[pl.ds(wid * _CPT, _CPT)], dst_v)
    plsc.subcore_barrier()

    def body(j, carry):
        pltpu.sync_copy(ones_v, acc.at[dst_v.at[j]], add=True)
        return carry

    lax.fori_loop(0, _CPT, body, 0)
    plsc.subcore_barrier()
    pltpu.sync_copy(acc.at[pl.ds(sid * _RPS, _RPS)],
                    out_hbm.at[cid, pl.ds(sid * _RPS, _RPS)])


def _make_deg_call():
    return pl.kernel(
        _deg_body,
        out_type=jax.ShapeDtypeStruct((2, _NPAD, 8), jnp.float32),
        mesh=_sc_mesh(),
        scratch_types=[
            pltpu.VMEM_SHARED((_NPAD, 8), jnp.float32),
            pltpu.VMEM((_CPT, _CHUNK), jnp.int32),
            pltpu.VMEM((_CHUNK, 8), jnp.float32),
        ],
    )


def _agg_body(y_hbm, src_hbm, dst_hbm, zeros_hbm, out_hbm,
              acc, src_v, dst_v, rows_v, sem0, sem1):
    cid = lax.axis_index("c")
    sid = lax.axis_index("s")
    wid = sid * 2 + cid
    pltpu.sync_copy(zeros_hbm.at[pl.ds(sid * _RPS, _RPS)],
                    acc.at[pl.ds(sid * _RPS, _RPS)])
    pltpu.sync_copy(src_hbm.at[pl.ds(wid * _CPT, _CPT)], src_v)
    pltpu.sync_copy(dst_hbm.at[pl.ds(wid * _CPT, _CPT)], dst_v)
    plsc.subcore_barrier()

    pltpu.async_copy(y_hbm.at[src_v.at[0]], rows_v.at[0], sem0)

    def body(i, carry):
        j0 = 2 * i
        j1 = 2 * i + 1

        @pl.when(j1 < _CPT)
        def _():
            pltpu.async_copy(y_hbm.at[src_v.at[j1]], rows_v.at[1], sem1)

        pltpu.make_async_copy(y_hbm.at[src_v.at[j0]], rows_v.at[0],
                              sem0).wait()
        pltpu.sync_copy(rows_v.at[0], acc.at[dst_v.at[j0]], add=True)

        @pl.when(j0 + 2 < _CPT)
        def _():
            pltpu.async_copy(y_hbm.at[src_v.at[j0 + 2]], rows_v.at[0], sem0)

        @pl.when(j1 < _CPT)
        def _():
            pltpu.make_async_copy(y_hbm.at[src_v.at[j1]], rows_v.at[1],
                                  sem1).wait()
            pltpu.sync_copy(rows_v.at[1], acc.at[dst_v.at[j1]], add=True)

        return carry

    lax.fori_loop(0, (_CPT + 1) // 2, body, 0)
    plsc.subcore_barrier()
    pltpu.sync_copy(acc.at[pl.ds(sid * _RPS, _RPS)],
                    out_hbm.at[cid, pl.ds(sid * _RPS, _RPS)])


def _make_agg_call():
    return pl.kernel(
        _agg_body,
        out_type=jax.ShapeDtypeStruct((2, _NPAD, _F), jnp.float32),
        mesh=_sc_mesh(),
        scratch_types=[
            pltpu.VMEM_SHARED((_NPAD, _F), jnp.float32),
            pltpu.VMEM((_CPT, _CHUNK), jnp.int32),
            pltpu.VMEM((_CPT, _CHUNK), jnp.int32),
            pltpu.VMEM((2, _CHUNK, _F), jnp.float32),
            pltpu.SemaphoreType.DMA,
            pltpu.SemaphoreType.DMA,
        ],
    )


def _pre_body(x_ref, w_ref, d0_ref, d1_ref, xw_ref, y_ref):
    d = d0_ref[:, 0:1] + d1_ref[:, 0:1] + 1.0
    dis = lax.rsqrt(d)
    xw = jnp.dot(x_ref[...], w_ref[...], preferred_element_type=jnp.float32)
    xw_ref[...] = xw
    y_ref[...] = xw * dis


def _make_pre_call():
    blk = 512
    return pl.pallas_call(
        _pre_body,
        grid=(_NPAD // blk,),
        in_specs=[
            pl.BlockSpec((blk, _F), lambda i: (i, 0)),
            pl.BlockSpec((_F, _F), lambda i: (0, 0)),
            pl.BlockSpec((blk, 8), lambda i: (i, 0)),
            pl.BlockSpec((blk, 8), lambda i: (i, 0)),
        ],
        out_specs=[pl.BlockSpec((blk, _F), lambda i: (i, 0))] * 2,
        out_shape=[jax.ShapeDtypeStruct((_NPAD, _F), jnp.float32)] * 2,
    )


def _mid_body(a0_ref, a1_ref, xw_ref, d0_ref, d1_ref, b_ref, g_ref, be_ref,
              w_ref, xwn_ref, yn_ref):
    d = d0_ref[:, 0:1] + d1_ref[:, 0:1] + 1.0
    dis = lax.rsqrt(d)
    t = dis * (a0_ref[...] + a1_ref[...]) + (1.0 / d) * xw_ref[...] + b_ref[...]
    mu = jnp.mean(t, axis=1, keepdims=True)
    tcen = t - mu
    var = jnp.mean(tcen * tcen, axis=1, keepdims=True)
    h = jnp.maximum(tcen * lax.rsqrt(var + 1e-5) * g_ref[...] + be_ref[...],
                    0.0)
    xwn = jnp.dot(h, w_ref[...], preferred_element_type=jnp.float32)
    xwn_ref[...] = xwn
    yn_ref[...] = xwn * dis


def _make_mid_call():
    blk = 512
    row = lambda i: (i, 0)
    fixed = lambda i: (0, 0)
    return pl.pallas_call(
        _mid_body,
        grid=(_NPAD // blk,),
        in_specs=[
            pl.BlockSpec((blk, _F), row),
            pl.BlockSpec((blk, _F), row),
            pl.BlockSpec((blk, _F), row),
            pl.BlockSpec((blk, 8), row),
            pl.BlockSpec((blk, 8), row),
            pl.BlockSpec((1, _F), fixed),
            pl.BlockSpec((1, _F), fixed),
            pl.BlockSpec((1, _F), fixed),
            pl.BlockSpec((_F, _F), fixed),
        ],
        out_specs=[pl.BlockSpec((blk, _F), row)] * 2,
        out_shape=[jax.ShapeDtypeStruct((_NPAD, _F), jnp.float32)] * 2,
    )


def _fin_body(a0_ref, a1_ref, xw_ref, d0_ref, d1_ref, b_ref, g_ref, be_ref,
              batch_ref, aW1_ref, ab1_ref, aW2_ref, ab2_ref,
              cW1_ref, cb1_ref, cW2_ref, cb2_ref,
              logits_ref, val_ref, sums, maxs, cnts):
    i = pl.program_id(0)
    blk = a0_ref.shape[0]

    d = d0_ref[:, 0:1] + d1_ref[:, 0:1] + 1.0
    dis = lax.rsqrt(d)
    t = dis * (a0_ref[...] + a1_ref[...]) + (1.0 / d) * xw_ref[...] + b_ref[...]
    mu = jnp.mean(t, axis=1, keepdims=True)
    tcen = t - mu
    var = jnp.mean(tcen * tcen, axis=1, keepdims=True)
    h = jnp.maximum(tcen * lax.rsqrt(var + 1e-5) * g_ref[...] + be_ref[...],
                    0.0)

    @pl.when(i == 0)
    def _():
        sums[...] = jnp.zeros_like(sums)
        maxs[...] = jnp.full_like(maxs, -1e30)
        cnts[...] = jnp.zeros_like(cnts)

    bvec = batch_ref[0, 0, :]
    onehot = (bvec[:, None] == lax.broadcasted_iota(jnp.int32, (blk, _NG), 1)
              ).astype(jnp.float32)
    dn = (((0,), (0,)), ((), ()))
    sums[...] += lax.dot_general(onehot, h, dn,
                                 preferred_element_type=jnp.float32)
    cnts[...] += lax.dot_general(onehot, jnp.ones_like(h), dn,
                                 preferred_element_type=jnp.float32)
    parts = []
    for g in range(_NG):
        m = jnp.where((bvec == g)[:, None], h, -1e30)
        parts.append(jnp.max(m, axis=0, keepdims=True))
    maxs[...] = jnp.maximum(maxs[...], jnp.concatenate(parts, axis=0))

    @pl.when(i == pl.num_programs(0) - 1)
    def _():
        cnt = cnts[...]
        mean = sums[...] / jnp.maximum(cnt, 1.0)
        xmax = jnp.where(cnt > 0, maxs[...], 0.0)
        emb = jnp.concatenate([mean, xmax], axis=1)
        a = jnp.tanh(jnp.dot(emb, aW1_ref[...],
                             preferred_element_type=jnp.float32)
                     + ab1_ref[...])
        logits_ref[...] = jnp.dot(a, aW2_ref[...],
                                  preferred_element_type=jnp.float32
                                  ) + ab2_ref[...]
        c = jnp.tanh(jnp.dot(emb, cW1_ref[...],
                             preferred_element_type=jnp.float32)
                     + cb1_ref[...])
        val_ref[...] = jnp.dot(c, cW2_ref[...],
                               preferred_element_type=jnp.float32
                               ) + cb2_ref[...]


def _make_fin_call():
    blk = 1024
    row = lambda i: (i, 0)
    fixed = lambda i: (0, 0)
    return pl.pallas_call(
        _fin_body,
        grid=(_NPAD // blk,),
        in_specs=[
            pl.BlockSpec((blk, _F), row),
            pl.BlockSpec((blk, _F), row),
            pl.BlockSpec((blk, _F), row),
            pl.BlockSpec((blk, 8), row),
            pl.BlockSpec((blk, 8), row),
            pl.BlockSpec((1, _F), fixed),
            pl.BlockSpec((1, _F), fixed),
            pl.BlockSpec((1, _F), fixed),
            pl.BlockSpec((1, 1, blk), lambda i: (i, 0, 0)),
            pl.BlockSpec((2 * _F, _F), fixed),
            pl.BlockSpec((1, _F), fixed),
            pl.BlockSpec((_F, 2 * _F), fixed),
            pl.BlockSpec((1, 2 * _F), fixed),
            pl.BlockSpec((2 * _F, _F), fixed),
            pl.BlockSpec((1, _F), fixed),
            pl.BlockSpec((_F, _F), fixed),
            pl.BlockSpec((1, _F), fixed),
        ],
        out_specs=[
            pl.BlockSpec((_NG, 2 * _F), fixed),
            pl.BlockSpec((_NG, _F), fixed),
        ],
        out_shape=[
            jax.ShapeDtypeStruct((_NG, 2 * _F), jnp.float32),
            jax.ShapeDtypeStruct((_NG, _F), jnp.float32),
        ],
        scratch_shapes=[
            pltpu.VMEM((_NG, _F), jnp.float32),
            pltpu.VMEM((_NG, _F), jnp.float32),
            pltpu.VMEM((_NG, _F), jnp.float32),
        ],
    )


def kernel(x, edge_index, batch, W1, b1, W2, b2, W3, b3, g1, be1, g2, be2,
           g3, be3, aW1, ab1, aW2, ab2, cW1, cb1, cW2, cb2):
    src = edge_index[0]
    dst = edge_index[1]

    pad = _EPAD - _E
    ar = jnp.arange(pad, dtype=jnp.int32)
    src2d = jnp.concatenate([src, ar % _N]).reshape(_NTILES * _CPT, _CHUNK)
    dst2d = jnp.concatenate([dst, _N + ar % _SINK]).reshape(
        _NTILES * _CPT, _CHUNK)

    xpad = jnp.zeros((_NPAD, _F), jnp.float32).at[:_N].set(x)
    batchpad = jnp.concatenate(
        [batch, jnp.full((_NPAD - _N,), _NG, jnp.int32)]).reshape(
            _NPAD // 1024, 1, 1024)

    ones8 = jnp.ones((_CHUNK, 8), jnp.float32)
    z8 = jnp.zeros((_NPAD, 8), jnp.float32)
    z128 = jnp.zeros((_NPAD, _F), jnp.float32)

    b1r, b2r, b3r = (v.reshape(1, _F) for v in (b1, b2, b3))
    g1r, g2r, g3r = (v.reshape(1, _F) for v in (g1, g2, g3))
    be1r, be2r, be3r = (v.reshape(1, _F) for v in (be1, be2, be3))
    ab1r = ab1.reshape(1, _F)
    cb1r = cb1.reshape(1, _F)
    aW2p = jnp.zeros((_F, 2 * _F), jnp.float32).at[:, :160].set(aW2)
    ab2p = jnp.zeros((1, 2 * _F), jnp.float32).at[0, :160].set(ab2)
    cW2p = jnp.zeros((_F, _F), jnp.float32).at[:, 0:1].set(cW2)
    cb2p = jnp.zeros((1, _F), jnp.float32).at[0, 0].set(cb2[0])

    deg_call = _make_deg_call()
    agg_call = _make_agg_call()
    pre_call = _make_pre_call()
    mid_call = _make_mid_call()
    fin_call = _make_fin_call()

    degp = deg_call(dst2d, ones8, z8)
    d0, d1 = degp[0], degp[1]

    xw1, y1 = pre_call(xpad, W1, d0, d1)
    agg1 = agg_call(y1, src2d, dst2d, z128)
    xw2, y2 = mid_call(agg1[0], agg1[1], xw1, d0, d1, b1r, g1r, be1r, W2)
    agg2 = agg_call(y2, src2d, dst2d, z128)
    xw3, y3 = mid_call(agg2[0], agg2[1], xw2, d0, d1, b2r, g2r, be2r, W3)
    agg3 = agg_call(y3, src2d, dst2d, z128)
    logits_pad, val2d = fin_call(agg3[0], agg3[1], xw3, d0, d1, b3r, g3r,
                                 be3r, batchpad, aW1, ab1r, aW2p, ab2p,
                                 cW1, cb1r, cW2p, cb2p)
    return logits_pad[:, :160], val2d[:, 0]


# speedup vs baseline: 16.2076x; 16.2076x over previous
import functools

import jax
import jax.numpy as jnp
from jax import lax
from jax.experimental import pallas as pl
from jax.experimental.pallas import tpu as pltpu
from jax.experimental.pallas import tpu_sc as plsc

_N = 10000
_E = 320000
_F = 128
_NG = 16
_NPAD = 10240
_CHUNK = 128
_NTILES = 32
_CPT = 80
_EPAD = _NTILES * _CPT * _CHUNK
_SINK = _NPAD - _N
_RPS = _NPAD // 16


def _sc_mesh():
    return plsc.VectorSubcoreMesh(core_axis_name="c", subcore_axis_name="s")


def _agg_body(y_hbm, src_hbm, dst_hbm, zeros_hbm, out_hbm,
              acc, src_v, dst_v, rows_v, sem0, sem1):
    cid = lax.axis_index("c")
    sid = lax.axis_index("s")
    wid = sid * 2 + cid
    pltpu.sync_copy(zeros_hbm.at[pl.ds(sid * _RPS, _RPS)],
                    acc.at[pl.ds(sid * _RPS, _RPS)])
    plsc.subcore_barrier()

    half = _CPT // 2
    for hf in range(2):
        base = wid * _CPT + hf * half
        pltpu.sync_copy(src_hbm.at[pl.ds(base, half)], src_v)
        pltpu.sync_copy(dst_hbm.at[pl.ds(base, half)], dst_v)
        pltpu.async_copy(y_hbm.at[src_v.at[0]], rows_v.at[0], sem0)

        def body(i, carry):
            j0 = 2 * i
            j1 = 2 * i + 1

            pltpu.async_copy(y_hbm.at[src_v.at[j1]], rows_v.at[1], sem1)
            pltpu.make_async_copy(y_hbm.at[src_v.at[j0]], rows_v.at[0],
                                  sem0).wait()
            pltpu.sync_copy(rows_v.at[0], acc.at[dst_v.at[j0]], add=True)

            @pl.when(j0 + 2 < half)
            def _():
                pltpu.async_copy(y_hbm.at[src_v.at[j0 + 2]], rows_v.at[0],
                                 sem0)

            pltpu.make_async_copy(y_hbm.at[src_v.at[j1]], rows_v.at[1],
                                  sem1).wait()
            pltpu.sync_copy(rows_v.at[1], acc.at[dst_v.at[j1]], add=True)
            return carry

        lax.fori_loop(0, half // 2, body, 0)
    plsc.subcore_barrier()
    pltpu.sync_copy(acc.at[pl.ds(sid * _RPS, _RPS)],
                    out_hbm.at[cid, pl.ds(sid * _RPS, _RPS)])


def _make_agg_call():
    return pl.kernel(
        _agg_body,
        out_type=jax.ShapeDtypeStruct((2, _NPAD, _F), jnp.float32),
        mesh=_sc_mesh(),
        scratch_types=[
            pltpu.VMEM_SHARED((_NPAD, _F), jnp.float32),
            pltpu.VMEM((_CPT // 2, _CHUNK), jnp.int32),
            pltpu.VMEM((_CPT // 2, _CHUNK), jnp.int32),
            pltpu.VMEM((2, _CHUNK, _F), jnp.float32),
            pltpu.SemaphoreType.DMA,
            pltpu.SemaphoreType.DMA,
        ],
    )


def _pre_body(x_ref, w_ref, d0_ref, d1_ref, xw_ref, y_ref):
    d = d0_ref[:, 0:1] + d1_ref[:, 0:1] + 1.0
    dis = lax.rsqrt(d)
    xw = jnp.dot(x_ref[...], w_ref[...], preferred_element_type=jnp.float32)
    xw_ref[...] = xw
    y_ref[...] = xw * dis


def _make_pre_call():
    blk = 512
    return pl.pallas_call(
        _pre_body,
        grid=(_NPAD // blk,),
        in_specs=[
            pl.BlockSpec((blk, _F), lambda i: (i, 0)),
            pl.BlockSpec((_F, _F), lambda i: (0, 0)),
            pl.BlockSpec((blk, _F), lambda i: (i, 0)),
            pl.BlockSpec((blk, _F), lambda i: (i, 0)),
        ],
        out_specs=[pl.BlockSpec((blk, _F), lambda i: (i, 0))] * 2,
        out_shape=[jax.ShapeDtypeStruct((_NPAD, _F), jnp.float32)] * 2,
    )


def _mid_body(a0_ref, a1_ref, xw_ref, d0_ref, d1_ref, b_ref, g_ref, be_ref,
              w_ref, xwn_ref, yn_ref):
    d = d0_ref[:, 0:1] + d1_ref[:, 0:1] + 1.0
    dis = lax.rsqrt(d)
    t = dis * (a0_ref[...] + a1_ref[...]) + (1.0 / d) * xw_ref[...] + b_ref[...]
    mu = jnp.mean(t, axis=1, keepdims=True)
    tcen = t - mu
    var = jnp.mean(tcen * tcen, axis=1, keepdims=True)
    h = jnp.maximum(tcen * lax.rsqrt(var + 1e-5) * g_ref[...] + be_ref[...],
                    0.0)
    xwn = jnp.dot(h, w_ref[...], preferred_element_type=jnp.float32)
    xwn_ref[...] = xwn
    yn_ref[...] = xwn * dis


def _make_mid_call():
    blk = 512
    row = lambda i: (i, 0)
    fixed = lambda i: (0, 0)
    return pl.pallas_call(
        _mid_body,
        grid=(_NPAD // blk,),
        in_specs=[
            pl.BlockSpec((blk, _F), row),
            pl.BlockSpec((blk, _F), row),
            pl.BlockSpec((blk, _F), row),
            pl.BlockSpec((blk, _F), row),
            pl.BlockSpec((blk, _F), row),
            pl.BlockSpec((1, _F), fixed),
            pl.BlockSpec((1, _F), fixed),
            pl.BlockSpec((1, _F), fixed),
            pl.BlockSpec((_F, _F), fixed),
        ],
        out_specs=[pl.BlockSpec((blk, _F), row)] * 2,
        out_shape=[jax.ShapeDtypeStruct((_NPAD, _F), jnp.float32)] * 2,
    )


def _fin_body(a0_ref, a1_ref, xw_ref, d0_ref, d1_ref, b_ref, g_ref, be_ref,
              batch_ref, aW1_ref, ab1_ref, aW2_ref, ab2_ref,
              cW1_ref, cb1_ref, cW2_ref, cb2_ref,
              logits_ref, val_ref, sums, maxs, cnts):
    i = pl.program_id(0)
    blk = a0_ref.shape[0]

    d = d0_ref[:, 0:1] + d1_ref[:, 0:1] + 1.0
    dis = lax.rsqrt(d)
    t = dis * (a0_ref[...] + a1_ref[...]) + (1.0 / d) * xw_ref[...] + b_ref[...]
    mu = jnp.mean(t, axis=1, keepdims=True)
    tcen = t - mu
    var = jnp.mean(tcen * tcen, axis=1, keepdims=True)
    h = jnp.maximum(tcen * lax.rsqrt(var + 1e-5) * g_ref[...] + be_ref[...],
                    0.0)

    @pl.when(i == 0)
    def _():
        sums[...] = jnp.zeros_like(sums)
        maxs[...] = jnp.full_like(maxs, -1e30)
        cnts[...] = jnp.zeros_like(cnts)

    bvec = batch_ref[0, 0, :]
    onehot = (bvec[:, None] == lax.broadcasted_iota(jnp.int32, (blk, _NG), 1)
              ).astype(jnp.float32)
    dn = (((0,), (0,)), ((), ()))
    sums[...] += lax.dot_general(onehot, h, dn,
                                 preferred_element_type=jnp.float32)
    cnts[...] += lax.dot_general(onehot, jnp.ones_like(h), dn,
                                 preferred_element_type=jnp.float32)
    parts = []
    for g in range(_NG):
        m = jnp.where(onehot[:, g:g + 1] > 0.0, h, -1e30)
        parts.append(jnp.max(m, axis=0, keepdims=True))
    maxs[...] = jnp.maximum(maxs[...], jnp.concatenate(parts, axis=0))

    @pl.when(i == pl.num_programs(0) - 1)
    def _():
        cnt = cnts[...]
        mean = sums[...] / jnp.maximum(cnt, 1.0)
        xmax = jnp.where(cnt > 0, maxs[...], 0.0)
        emb = jnp.concatenate([mean, xmax], axis=1)
        a = jnp.tanh(jnp.dot(emb, aW1_ref[...],
                             preferred_element_type=jnp.float32)
                     + ab1_ref[...])
        logits_ref[...] = jnp.dot(a, aW2_ref[...],
                                  preferred_element_type=jnp.float32
                                  ) + ab2_ref[...]
        c = jnp.tanh(jnp.dot(emb, cW1_ref[...],
                             preferred_element_type=jnp.float32)
                     + cb1_ref[...])
        val_ref[...] = jnp.dot(c, cW2_ref[...],
                               preferred_element_type=jnp.float32
                               ) + cb2_ref[...]


def _make_fin_call():
    blk = 1024
    row = lambda i: (i, 0)
    fixed = lambda i: (0, 0)
    return pl.pallas_call(
        _fin_body,
        grid=(_NPAD // blk,),
        in_specs=[
            pl.BlockSpec((blk, _F), row),
            pl.BlockSpec((blk, _F), row),
            pl.BlockSpec((blk, _F), row),
            pl.BlockSpec((blk, _F), row),
            pl.BlockSpec((blk, _F), row),
            pl.BlockSpec((1, _F), fixed),
            pl.BlockSpec((1, _F), fixed),
            pl.BlockSpec((1, _F), fixed),
            pl.BlockSpec((1, 1, blk), lambda i: (i, 0, 0)),
            pl.BlockSpec((2 * _F, _F), fixed),
            pl.BlockSpec((1, _F), fixed),
            pl.BlockSpec((_F, 2 * _F), fixed),
            pl.BlockSpec((1, 2 * _F), fixed),
            pl.BlockSpec((2 * _F, _F), fixed),
            pl.BlockSpec((1, _F), fixed),
            pl.BlockSpec((_F, _F), fixed),
            pl.BlockSpec((1, _F), fixed),
        ],
        out_specs=[
            pl.BlockSpec((_NG, 2 * _F), fixed),
            pl.BlockSpec((_NG, _F), fixed),
        ],
        out_shape=[
            jax.ShapeDtypeStruct((_NG, 2 * _F), jnp.float32),
            jax.ShapeDtypeStruct((_NG, _F), jnp.float32),
        ],
        scratch_shapes=[
            pltpu.VMEM((_NG, _F), jnp.float32),
            pltpu.VMEM((_NG, _F), jnp.float32),
            pltpu.VMEM((_NG, _F), jnp.float32),
        ],
    )


def kernel(x, edge_index, batch, W1, b1, W2, b2, W3, b3, g1, be1, g2, be2,
           g3, be3, aW1, ab1, aW2, ab2, cW1, cb1, cW2, cb2):
    src = edge_index[0]
    dst = edge_index[1]

    pad = _EPAD - _E
    ar = jnp.arange(pad, dtype=jnp.int32)
    src2d = jnp.concatenate([src, ar % _N]).reshape(_NTILES * _CPT, _CHUNK)
    dst2d = jnp.concatenate([dst, _N + ar % _SINK]).reshape(
        _NTILES * _CPT, _CHUNK)

    xpad = jnp.zeros((_NPAD, _F), jnp.float32).at[:_N].set(x)
    batchpad = jnp.concatenate(
        [batch, jnp.full((_NPAD - _N,), _NG, jnp.int32)]).reshape(
            _NPAD // 1024, 1, 1024)

    ones_y = jnp.ones((_NPAD, _F), jnp.float32)
    z128 = jnp.zeros((_NPAD, _F), jnp.float32)

    b1r, b2r, b3r = (v.reshape(1, _F) for v in (b1, b2, b3))
    g1r, g2r, g3r = (v.reshape(1, _F) for v in (g1, g2, g3))
    be1r, be2r, be3r = (v.reshape(1, _F) for v in (be1, be2, be3))
    ab1r = ab1.reshape(1, _F)
    cb1r = cb1.reshape(1, _F)
    aW2p = jnp.zeros((_F, 2 * _F), jnp.float32).at[:, :160].set(aW2)
    ab2p = jnp.zeros((1, 2 * _F), jnp.float32).at[0, :160].set(ab2)
    cW2p = jnp.zeros((_F, _F), jnp.float32).at[:, 0:1].set(cW2)
    cb2p = jnp.zeros((1, _F), jnp.float32).at[0, 0].set(cb2[0])

    agg_call = _make_agg_call()
    pre_call = _make_pre_call()
    mid_call = _make_mid_call()
    fin_call = _make_fin_call()

    degp = agg_call(ones_y, src2d, dst2d, z128)
    d0, d1 = degp[0], degp[1]

    xw1, y1 = pre_call(xpad, W1, d0, d1)
    agg1 = agg_call(y1, src2d, dst2d, z128)
    xw2, y2 = mid_call(agg1[0], agg1[1], xw1, d0, d1, b1r, g1r, be1r, W2)
    agg2 = agg_call(y2, src2d, dst2d, z128)
    xw3, y3 = mid_call(agg2[0], agg2[1], xw2, d0, d1, b2r, g2r, be2r, W3)
    agg3 = agg_call(y3, src2d, dst2d, z128)
    logits_pad, val2d = fin_call(agg3[0], agg3[1], xw3, d0, d1, b3r, g3r,
                                 be3r, batchpad, aW1, ab1r, aW2p, ab2p,
                                 cW1, cb1r, cW2p, cb2p)
    return logits_pad[:, :160], val2d[:, 0]


# speedup vs baseline: 19.7484x; 1.2185x over previous
import functools

import numpy as np

import jax
import jax.numpy as jnp
from jax import lax
from jax.experimental import pallas as pl
from jax.experimental.pallas import tpu as pltpu
from jax.experimental.pallas import tpu_sc as plsc

_N = 10000
_E = 320000
_F = 128
_NG = 16
_NPAD = 10240
_CHUNK = 128
_NTILES = 32
_CPT = 80
_EPAD = _NTILES * _CPT * _CHUNK
_SINK = _NPAD - _N
_RPS = _NPAD // 16


def _sc_mesh():
    return plsc.VectorSubcoreMesh(core_axis_name="c", subcore_axis_name="s")


_DG = 128


def _deg_body(ei_hbm, ones_hbm, zeros_hbm, out_hbm, acc, dst_v, ones_v):
    cid = lax.axis_index("c")
    sid = lax.axis_index("s")
    wid = sid * 2 + cid
    pltpu.sync_copy(zeros_hbm, ones_v)
    for k in range(_RPS // _CHUNK):
        pltpu.sync_copy(ones_v,
                        acc.at[pl.ds(sid * _RPS + k * _CHUNK, _CHUNK)])
    pltpu.sync_copy(ones_hbm, ones_v)
    plsc.subcore_barrier()

    half = _CPT // 2
    for hf in range(2):
        pltpu.sync_copy(ei_hbm.at[1, pl.ds(wid * _CPT + hf * half, half)],
                        dst_v)

        def body(j, carry):
            pltpu.sync_copy(ones_v, acc.at[dst_v.at[j]], add=True)
            return carry

        lax.fori_loop(0, half, body, 0)
    plsc.subcore_barrier()
    pltpu.sync_copy(acc.at[pl.ds(sid * _RPS, _RPS)],
                    out_hbm.at[cid, pl.ds(sid * _RPS, _RPS)])


def _make_deg_call():
    return pl.kernel(
        _deg_body,
        out_type=jax.ShapeDtypeStruct((2, _NPAD, _DG), jnp.float32),
        mesh=_sc_mesh(),
        scratch_types=[
            pltpu.VMEM_SHARED((_NPAD, _DG), jnp.float32),
            pltpu.VMEM((_CPT // 2, _CHUNK), jnp.int32),
            pltpu.VMEM((_CHUNK, _DG), jnp.float32),
        ],
    )


def _agg_body(y_hbm, ei_hbm, zeros_hbm, out_hbm,
              acc, src_v, dst_v, rows_v, sg0, sg1):
    cid = lax.axis_index("c")
    sid = lax.axis_index("s")
    wid = sid * 2 + cid
    pltpu.sync_copy(zeros_hbm, rows_v.at[0])
    for k in range(_RPS // _CHUNK):
        pltpu.sync_copy(rows_v.at[0],
                        acc.at[pl.ds(sid * _RPS + k * _CHUNK, _CHUNK)])
    plsc.subcore_barrier()

    half = _CPT // 2
    for hf in range(2):
        base = wid * _CPT + hf * half
        pltpu.sync_copy(ei_hbm.at[0, pl.ds(base, half)], src_v)
        pltpu.sync_copy(ei_hbm.at[1, pl.ds(base, half)], dst_v)
        pltpu.async_copy(y_hbm.at[src_v.at[0]], rows_v.at[0], sg0)

        def body(i, carry):
            j0 = 2 * i
            j1 = 2 * i + 1

            pltpu.async_copy(y_hbm.at[src_v.at[j1]], rows_v.at[1], sg1)
            pltpu.make_async_copy(y_hbm.at[src_v.at[j0]], rows_v.at[0],
                                  sg0).wait()
            pltpu.sync_copy(rows_v.at[0], acc.at[dst_v.at[j0]], add=True)

            @pl.when(j0 + 2 < half)
            def _():
                pltpu.async_copy(y_hbm.at[src_v.at[j0 + 2]], rows_v.at[0],
                                 sg0)

            pltpu.make_async_copy(y_hbm.at[src_v.at[j1]], rows_v.at[1],
                                  sg1).wait()
            pltpu.sync_copy(rows_v.at[1], acc.at[dst_v.at[j1]], add=True)
            return carry

        lax.fori_loop(0, half // 2, body, 0)
    plsc.subcore_barrier()
    pltpu.sync_copy(acc.at[pl.ds(sid * _RPS, _RPS)],
                    out_hbm.at[cid, pl.ds(sid * _RPS, _RPS)])


def _make_agg_call():
    return pl.kernel(
        _agg_body,
        out_type=jax.ShapeDtypeStruct((2, _NPAD, _F), jnp.float32),
        mesh=_sc_mesh(),
        scratch_types=[
            pltpu.VMEM_SHARED((_NPAD, _F), jnp.float32),
            pltpu.VMEM((_CPT // 2, _CHUNK), jnp.int32),
            pltpu.VMEM((_CPT // 2, _CHUNK), jnp.int32),
            pltpu.VMEM((2, _CHUNK, _F), jnp.float32),
            pltpu.SemaphoreType.DMA,
            pltpu.SemaphoreType.DMA,
        ],
    )


def _mm_body(x_ref, w_ref, xw_ref):
    xw_ref[...] = jnp.dot(x_ref[...], w_ref[...],
                          preferred_element_type=jnp.float32)


def _make_mm_call():
    blk = 1000
    return pl.pallas_call(
        _mm_body,
        grid=(_N // blk,),
        in_specs=[
            pl.BlockSpec((blk, _F), lambda i: (i, 0)),
            pl.BlockSpec((_F, _F), lambda i: (0, 0)),
        ],
        out_specs=[pl.BlockSpec((blk, _F), lambda i: (i, 0))],
        out_shape=[jax.ShapeDtypeStruct((_N, _F), jnp.float32)],
    )


def _pre_body(xw_ref, dp_ref, y_ref, dis_ref):
    d = dp_ref[0, :, 0:1] + dp_ref[1, :, 0:1] + 1.0
    dis = lax.rsqrt(d)
    y_ref[...] = xw_ref[...] * dis
    dis_ref[...] = jnp.broadcast_to(dis, dis_ref.shape)


def _make_pre_call():
    blk = 1000
    return pl.pallas_call(
        _pre_body,
        grid=(_N // blk,),
        in_specs=[
            pl.BlockSpec((blk, _F), lambda i: (i, 0)),
            pl.BlockSpec((2, blk, _DG), lambda i: (0, i, 0)),
        ],
        out_specs=[
            pl.BlockSpec((blk, _F), lambda i: (i, 0)),
            pl.BlockSpec((blk, 8), lambda i: (i, 0)),
        ],
        out_shape=[
            jax.ShapeDtypeStruct((_N, _F), jnp.float32),
            jax.ShapeDtypeStruct((_N, 8), jnp.float32),
        ],
    )


def _mid_body(ap_ref, xw_ref, dis_ref, b_ref, g_ref, be_ref,
              w_ref, xwn_ref, yn_ref):
    dis = dis_ref[:, 0:1]
    t = (dis * (ap_ref[0] + ap_ref[1])
         + (dis * dis) * xw_ref[...] + b_ref[...])
    mu = jnp.mean(t, axis=1, keepdims=True)
    tcen = t - mu
    var = jnp.mean(tcen * tcen, axis=1, keepdims=True)
    h = jnp.maximum(tcen * lax.rsqrt(var + 1e-5) * g_ref[...] + be_ref[...],
                    0.0)
    xwn = jnp.dot(h, w_ref[...], preferred_element_type=jnp.float32)
    xwn_ref[...] = xwn
    yn_ref[...] = xwn * dis


def _make_mid_call():
    blk = 1000
    row = lambda i: (i, 0)
    fixed = lambda i: (0, 0)
    return pl.pallas_call(
        _mid_body,
        grid=(_N // blk,),
        in_specs=[
            pl.BlockSpec((2, blk, _F), lambda i: (0, i, 0)),
            pl.BlockSpec((blk, _F), row),
            pl.BlockSpec((blk, 8), row),
            pl.BlockSpec((1, _F), fixed),
            pl.BlockSpec((1, _F), fixed),
            pl.BlockSpec((1, _F), fixed),
            pl.BlockSpec((_F, _F), fixed),
        ],
        out_specs=[pl.BlockSpec((blk, _F), row)] * 2,
        out_shape=[jax.ShapeDtypeStruct((_N, _F), jnp.float32)] * 2,
    )


def _fin_body(ap_ref, xw_ref, dis_ref, b_ref, g_ref, be_ref,
              batch_ref, aW1_ref, ab1_ref, aW2_ref, ab2_ref,
              cW1_ref, cb1_ref, cW2_ref, cb2_ref,
              logits_ref, val_ref, sums, maxs, cnts):
    i = pl.program_id(0)
    blk = xw_ref.shape[0]

    dis = dis_ref[:, 0:1]
    t = (dis * (ap_ref[0] + ap_ref[1])
         + (dis * dis) * xw_ref[...] + b_ref[...])
    mu = jnp.mean(t, axis=1, keepdims=True)
    tcen = t - mu
    var = jnp.mean(tcen * tcen, axis=1, keepdims=True)
    h = jnp.maximum(tcen * lax.rsqrt(var + 1e-5) * g_ref[...] + be_ref[...],
                    0.0)

    @pl.when(i == 0)
    def _():
        sums[...] = jnp.zeros_like(sums)
        maxs[...] = jnp.full_like(maxs, -1e30)
        cnts[...] = jnp.zeros_like(cnts)

    bvec = batch_ref[0, 0, :]
    onehot = (bvec[:, None] == lax.broadcasted_iota(jnp.int32, (blk, _NG), 1)
              ).astype(jnp.float32)
    dn = (((0,), (0,)), ((), ()))
    sums[...] += lax.dot_general(onehot, h, dn,
                                 preferred_element_type=jnp.float32)
    cnts[...] += lax.dot_general(onehot, jnp.ones_like(h), dn,
                                 preferred_element_type=jnp.float32)
    parts = []
    for g in range(_NG):
        parts.append(jnp.max(h * onehot[:, g:g + 1], axis=0, keepdims=True))
    maxs[...] = jnp.maximum(maxs[...], jnp.concatenate(parts, axis=0))

    @pl.when(i == pl.num_programs(0) - 1)
    def _():
        cnt = cnts[...]
        mean = sums[...] / jnp.maximum(cnt, 1.0)
        xmax = jnp.where(cnt > 0, maxs[...], 0.0)
        emb = jnp.concatenate([mean, xmax], axis=1)
        a = jnp.tanh(jnp.dot(emb, aW1_ref[...],
                             preferred_element_type=jnp.float32)
                     + ab1_ref[...])
        logits_ref[...] = jnp.dot(a, aW2_ref[...],
                                  preferred_element_type=jnp.float32
                                  ) + ab2_ref[...]
        c = jnp.tanh(jnp.dot(emb, cW1_ref[...],
                             preferred_element_type=jnp.float32)
                     + cb1_ref[...])
        val_ref[...] = jnp.dot(c, cW2_ref[...],
                               preferred_element_type=jnp.float32
                               ) + cb2_ref[...]


def _make_fin_call():
    blk = 1000
    row = lambda i: (i, 0)
    fixed = lambda i: (0, 0)
    return pl.pallas_call(
        _fin_body,
        grid=(_N // blk,),
        in_specs=[
            pl.BlockSpec((2, blk, _F), lambda i: (0, i, 0)),
            pl.BlockSpec((blk, _F), row),
            pl.BlockSpec((blk, 8), row),
            pl.BlockSpec((1, _F), fixed),
            pl.BlockSpec((1, _F), fixed),
            pl.BlockSpec((1, _F), fixed),
            pl.BlockSpec((1, 1, 1000), lambda i: (i, 0, 0)),
            pl.BlockSpec((2 * _F, _F), fixed),
            pl.BlockSpec((1, _F), fixed),
            pl.BlockSpec((_F, 2 * _F), fixed),
            pl.BlockSpec((1, 2 * _F), fixed),
            pl.BlockSpec((2 * _F, _F), fixed),
            pl.BlockSpec((1, _F), fixed),
            pl.BlockSpec((_F, _F), fixed),
            pl.BlockSpec((1, _F), fixed),
        ],
        out_specs=[
            pl.BlockSpec((_NG, 2 * _F), fixed),
            pl.BlockSpec((_NG, _F), fixed),
        ],
        out_shape=[
            jax.ShapeDtypeStruct((_NG, 2 * _F), jnp.float32),
            jax.ShapeDtypeStruct((_NG, _F), jnp.float32),
        ],
        scratch_shapes=[
            pltpu.VMEM((_NG, _F), jnp.float32),
            pltpu.VMEM((_NG, _F), jnp.float32),
            pltpu.VMEM((_NG, _F), jnp.float32),
        ],
    )


def kernel(x, edge_index, batch, W1, b1, W2, b2, W3, b3, g1, be1, g2, be2,
           g3, be3, aW1, ab1, aW2, ab2, cW1, cb1, cW2, cb2):
    src = edge_index[0]
    dst = edge_index[1]

    pad = _EPAD - _E
    ar = np.arange(pad, dtype=np.int32)
    ei_pad = jnp.asarray(np.stack([ar % _N, _N + ar % _SINK]))
    ei2d = jnp.concatenate([edge_index, ei_pad], axis=1).reshape(
        2, _NTILES * _CPT, _CHUNK)

    batch3d = batch.reshape(_N // 1000, 1, 1000)
    ones_c = jnp.ones((_CHUNK, _DG), jnp.float32)
    zs = jnp.zeros((_CHUNK, _F), jnp.float32)

    b1r, b2r, b3r = (v.reshape(1, _F) for v in (b1, b2, b3))
    g1r, g2r, g3r = (v.reshape(1, _F) for v in (g1, g2, g3))
    be1r, be2r, be3r = (v.reshape(1, _F) for v in (be1, be2, be3))
    ab1r = ab1.reshape(1, _F)
    cb1r = cb1.reshape(1, _F)
    aW2p = jnp.zeros((_F, 2 * _F), jnp.float32).at[:, :160].set(aW2)
    ab2p = jnp.zeros((1, 2 * _F), jnp.float32).at[0, :160].set(ab2)
    cW2p = jnp.zeros((_F, _F), jnp.float32).at[:, 0:1].set(cW2)
    cb2p = jnp.zeros((1, _F), jnp.float32).at[0, 0].set(cb2[0])

    deg_call = _make_deg_call()
    agg_call = _make_agg_call()
    mm_call = _make_mm_call()
    pre_call = _make_pre_call()
    mid_call = _make_mid_call()
    fin_call = _make_fin_call()

    xw1, = mm_call(x, W1)
    degp = deg_call(ei2d, ones_c, zs)

    y1, dis8 = pre_call(xw1, degp)
    agg1 = agg_call(y1, ei2d, zs)
    xw2, y2 = mid_call(agg1, xw1, dis8, b1r, g1r, be1r, W2)
    agg2 = agg_call(y2, ei2d, zs)
    xw3, y3 = mid_call(agg2, xw2, dis8, b2r, g2r, be2r, W3)
    agg3 = agg_call(y3, ei2d, zs)
    logits_pad, val2d = fin_call(agg3, xw3, dis8, b3r, g3r,
                                 be3r, batch3d, aW1, ab1r, aW2p, ab2p,
                                 cW1, cb1r, cW2p, cb2p)
    return logits_pad[:, :160], val2d[:, 0]
